# baseline (speedup 1.0000x reference)
import functools

import jax
import jax.numpy as jnp
from jax import lax
from jax.experimental import pallas as pl
from jax.experimental.pallas import tpu as pltpu
from jax.experimental.pallas import tpu_sc as plsc

B = 16384
L = 50
D = 64
LANES = 16
SENTINEL = -1


def _sc_body(ncores, chunk, idxT1, len1, idxT2, len2, emb, out1, out2,
             idxT_v, len_v, accA, accB, semA, semB):
    ngroups = chunk // LANES
    wid = lax.axis_index("s") * ncores + lax.axis_index("c")
    base = wid * chunk

    def _idx(j):
        return plsc.Indices(idxT_v.at[j], ignored_value=SENTINEL)

    def do_side(idxT_hbm, len_hbm, out):
        pltpu.sync_copy(idxT_hbm.at[:, pl.ds(base, chunk)], idxT_v)
        pltpu.sync_copy(len_hbm.at[pl.ds(base, chunk)], len_v)

        def mask_body(j, c):
            for g in range(ngroups):
                sl = pl.ds(g * LANES, LANES)
                lenv = len_v[sl]
                cj = idxT_v[j, sl]
                idxT_v[j, sl] = jnp.where(j < lenv, cj, SENTINEL)
            return c
        lax.fori_loop(1, L, mask_body, 0)

        def zero_body(e, c):
            for k in range(D // LANES):
                accB[e, pl.ds(k * LANES, LANES)] = jnp.zeros((LANES,),
                                                             jnp.float32)
            return c
        lax.fori_loop(0, chunk, zero_body, 0)

        pltpu.async_copy(emb.at[idxT_v.at[0]], accA, semA)
        pltpu.async_copy(emb.at[_idx(1)], accB, semB, add=True)

        def pair_body(p, c):
            j0 = 2 * p
            pltpu.make_async_copy(emb.at[_idx(j0)], accA, semA).wait()
            pltpu.async_copy(emb.at[_idx(j0)], accA, semA, add=True)
            pltpu.make_async_copy(emb.at[_idx(j0 + 1)], accB, semB).wait()
            pltpu.async_copy(emb.at[_idx(j0 + 1)], accB, semB, add=True)
            return c
        lax.fori_loop(1, L // 2, pair_body, 0)

        pltpu.make_async_copy(emb.at[_idx(0)], accA, semA).wait()
        pltpu.make_async_copy(emb.at[_idx(1)], accB, semB).wait()

        def sum_body(e, c):
            for k in range(D // LANES):
                sl = pl.ds(k * LANES, LANES)
                accA[e, sl] = accA[e, sl] + accB[e, sl]
            return c
        lax.fori_loop(0, chunk, sum_body, 0)

        pltpu.sync_copy(accA, out.at[pl.ds(base, chunk), :])

    do_side(idxT1, len1, out1)
    do_side(idxT2, len2, out2)


def _tr_body(x_ref, o_ref):
    o_ref[...] = x_ref[...].T


def _cos_body(s1_ref, s2_ref, l1_ref, l2_ref, o_ref):
    a = s1_ref[...]
    b = s2_ref[...]
    d1 = jnp.maximum(l1_ref[...].astype(jnp.float32), 1.0)
    d2 = jnp.maximum(l2_ref[...].astype(jnp.float32), 1.0)
    dot = jnp.sum(a * b, axis=1, keepdims=True)
    n1 = jnp.sum(a * a, axis=1, keepdims=True)
    n2 = jnp.sum(b * b, axis=1, keepdims=True)
    g = dot / (d1 * d2)
    an = jnp.sqrt(n1) / d1
    bn = jnp.sqrt(n2) / d2
    o_ref[...] = g / jnp.maximum(an * bn, 1e-8)


def kernel(g_idxs1, g_lengths1, g_idxs2, g_lengths2, emb):
    info = plsc.get_sparse_core_info()
    nw = info.num_cores * info.num_subcores
    chunk = B // nw

    idxT1 = g_idxs1.T.astype(jnp.int32)
    idxT2 = g_idxs2.T.astype(jnp.int32)
    len1 = g_lengths1.astype(jnp.int32)
    len2 = g_lengths2.astype(jnp.int32)

    mesh = plsc.VectorSubcoreMesh(core_axis_name="c", subcore_axis_name="s")
    f32 = jnp.float32
    sc_fn = pl.kernel(
        functools.partial(_sc_body, info.num_cores, chunk),
        compiler_params=pltpu.CompilerParams(use_tc_tiling_on_sc=False),
        out_type=(
            jax.ShapeDtypeStruct((B, D), f32),
            jax.ShapeDtypeStruct((B, D), f32),
        ),
        mesh=mesh,
        scratch_types=[
            pltpu.VMEM((L, chunk), jnp.int32),
            pltpu.VMEM((chunk,), jnp.int32),
            pltpu.VMEM((chunk, D), f32),
            pltpu.VMEM((chunk, D), f32),
            pltpu.SemaphoreType.DMA,
            pltpu.SemaphoreType.DMA,
        ],
    )
    vocab = emb.shape[0]
    tblk = 4096
    emb128 = pl.pallas_call(
        _tr_body,
        grid=(pl.cdiv(vocab, tblk),),
        in_specs=[pl.BlockSpec((D, tblk), lambda i: (0, i))],
        out_specs=pl.BlockSpec((tblk, D), lambda i: (i, 0)),
        out_shape=jax.ShapeDtypeStruct((vocab, D), f32),
    )(emb.T)

    s1, s2 = sc_fn(idxT1, len1, idxT2, len2, emb128)

    blk = 2048
    cos = pl.pallas_call(
        _cos_body,
        grid=(B // blk,),
        in_specs=[
            pl.BlockSpec((blk, D), lambda i: (i, 0)),
            pl.BlockSpec((blk, D), lambda i: (i, 0)),
            pl.BlockSpec((blk, 1), lambda i: (i, 0)),
            pl.BlockSpec((blk, 1), lambda i: (i, 0)),
        ],
        out_specs=pl.BlockSpec((blk, 1), lambda i: (i, 0)),
        out_shape=jax.ShapeDtypeStruct((B, 1), f32),
    )(s1, s2, len1.reshape(B, 1), len2.reshape(B, 1))
    return cos.reshape(B)


# speedup vs baseline: 1.7113x; 1.7113x over previous
import functools

import jax
import jax.numpy as jnp
from jax import lax
from jax.experimental import pallas as pl
from jax.experimental.pallas import tpu as pltpu
from jax.experimental.pallas import tpu_sc as plsc

B = 16384
L = 50
D = 64
LANES = 16
SENTINEL = -1


def _sc_body(ncores, chunk, idxT1, len1, idxT2, len2, emb, out1, out2,
             idxT_v, len_v, accA, accB, semA, semB):
    ngroups = chunk // LANES
    wid = lax.axis_index("s") * ncores + lax.axis_index("c")
    base = wid * chunk

    def _idx(j):
        return plsc.Indices(idxT_v.at[j], ignored_value=SENTINEL)

    def do_side(idxT_hbm, len_hbm, out):
        pltpu.sync_copy(idxT_hbm.at[:, pl.ds(base, chunk)], idxT_v)
        pltpu.sync_copy(len_hbm.at[pl.ds(base, chunk)], len_v)

        def mask_body(j, c):
            for g in range(ngroups):
                sl = pl.ds(g * LANES, LANES)
                lenv = len_v[sl]
                cj = idxT_v[j, sl]
                idxT_v[j, sl] = jnp.where(j < lenv, cj * 2, SENTINEL)
            return c
        lax.fori_loop(0, L, mask_body, 0)

        def zero_body(e, c):
            for k in range(D // LANES):
                accB[e, pl.ds(k * LANES, LANES)] = jnp.zeros((LANES,),
                                                             jnp.float32)
            return c
        lax.fori_loop(0, chunk, zero_body, 0)

        pltpu.async_copy(emb.at[idxT_v.at[0]], accA, semA)
        pltpu.async_copy(emb.at[_idx(1)], accB, semB, add=True)

        def pair_body(p, c):
            j0 = 2 * p
            pltpu.make_async_copy(emb.at[_idx(j0)], accA, semA).wait()
            pltpu.async_copy(emb.at[_idx(j0)], accA, semA, add=True)
            pltpu.make_async_copy(emb.at[_idx(j0 + 1)], accB, semB).wait()
            pltpu.async_copy(emb.at[_idx(j0 + 1)], accB, semB, add=True)
            return c
        lax.fori_loop(1, L // 2, pair_body, 0)

        pltpu.make_async_copy(emb.at[_idx(0)], accA, semA).wait()
        pltpu.make_async_copy(emb.at[_idx(1)], accB, semB).wait()

        def sum_body(e, c):
            for k in range(D // LANES):
                sl = pl.ds(k * LANES, LANES)
                accA[e, sl] = accA[e, sl] + accB[e, sl]
            return c
        lax.fori_loop(0, chunk, sum_body, 0)

        pltpu.sync_copy(accA, out.at[pl.ds(base, chunk), :])

    do_side(idxT1, len1, out1)
    do_side(idxT2, len2, out2)


def _tr_body(x_ref, o_ref):
    o_ref[:, 0:D] = x_ref[...].T


def _cos_body(s1_ref, s2_ref, l1_ref, l2_ref, o_ref):
    a = s1_ref[...]
    b = s2_ref[...]
    d1 = jnp.maximum(l1_ref[...].astype(jnp.float32), 1.0)
    d2 = jnp.maximum(l2_ref[...].astype(jnp.float32), 1.0)
    dot = jnp.sum(a * b, axis=1, keepdims=True)
    n1 = jnp.sum(a * a, axis=1, keepdims=True)
    n2 = jnp.sum(b * b, axis=1, keepdims=True)
    g = dot / (d1 * d2)
    an = jnp.sqrt(n1) / d1
    bn = jnp.sqrt(n2) / d2
    o_ref[...] = g / jnp.maximum(an * bn, 1e-8)


def kernel(g_idxs1, g_lengths1, g_idxs2, g_lengths2, emb):
    info = plsc.get_sparse_core_info()
    nw = info.num_cores * info.num_subcores
    chunk = B // nw

    idxT1 = g_idxs1.T.astype(jnp.int32)
    idxT2 = g_idxs2.T.astype(jnp.int32)
    len1 = g_lengths1.astype(jnp.int32)
    len2 = g_lengths2.astype(jnp.int32)

    mesh = plsc.VectorSubcoreMesh(core_axis_name="c", subcore_axis_name="s")
    f32 = jnp.float32
    sc_fn = pl.kernel(
        functools.partial(_sc_body, info.num_cores, chunk),
        compiler_params=pltpu.CompilerParams(use_tc_tiling_on_sc=False),
        out_type=(
            jax.ShapeDtypeStruct((B, D), f32),
            jax.ShapeDtypeStruct((B, D), f32),
        ),
        mesh=mesh,
        scratch_types=[
            pltpu.VMEM((L, chunk), jnp.int32),
            pltpu.VMEM((chunk,), jnp.int32),
            pltpu.VMEM((chunk, D), f32),
            pltpu.VMEM((chunk, D), f32),
            pltpu.SemaphoreType.DMA,
            pltpu.SemaphoreType.DMA,
        ],
    )
    vocab = emb.shape[0]
    tblk = 4096
    emb128 = pl.pallas_call(
        _tr_body,
        grid=(pl.cdiv(vocab, tblk),),
        in_specs=[pl.BlockSpec((D, tblk), lambda i: (0, i))],
        out_specs=pl.BlockSpec((tblk, 2 * D), lambda i: (i, 0)),
        out_shape=jax.ShapeDtypeStruct((vocab, 2 * D), f32),
    )(emb.T)
    emb2 = emb128.reshape(2 * vocab, D)

    s1, s2 = sc_fn(idxT1, len1, idxT2, len2, emb2)

    blk = 2048
    cos = pl.pallas_call(
        _cos_body,
        grid=(B // blk,),
        in_specs=[
            pl.BlockSpec((blk, D), lambda i: (i, 0)),
            pl.BlockSpec((blk, D), lambda i: (i, 0)),
            pl.BlockSpec((blk, 1), lambda i: (i, 0)),
            pl.BlockSpec((blk, 1), lambda i: (i, 0)),
        ],
        out_specs=pl.BlockSpec((blk, 1), lambda i: (i, 0)),
        out_shape=jax.ShapeDtypeStruct((B, 1), f32),
    )(s1, s2, len1.reshape(B, 1), len2.reshape(B, 1))
    return cos.reshape(B)


# speedup vs baseline: 2.0164x; 1.1783x over previous
import functools

import jax
import jax.numpy as jnp
from jax import lax
from jax.experimental import pallas as pl
from jax.experimental.pallas import tpu as pltpu
from jax.experimental.pallas import tpu_sc as plsc

B = 16384
L = 50
D = 64
LANES = 16
SENTINEL = -1


def _sc_body(ncores, chunk, half, idxT1, len1, idxT2, len2, emb, out1, out2,
             idxT_v, len_v, accA, accB, semA, semB):
    ngroups = chunk // LANES
    wid = lax.axis_index("s") * ncores + lax.axis_index("c")
    base = wid * chunk

    def _idx(j):
        return plsc.Indices(idxT_v.at[j], ignored_value=SENTINEL)

    def do_side(idxT_hbm, len_hbm, out):
        pltpu.sync_copy(idxT_hbm.at[:, pl.ds(base, chunk)], idxT_v)
        pltpu.sync_copy(len_hbm.at[pl.ds(base, chunk)], len_v)

        def mask_body(j, c):
            for g in range(ngroups):
                sl = pl.ds(g * LANES, LANES)
                lenv = len_v[sl]
                cj = idxT_v[j, sl]
                adj = jnp.where(cj < half, cj * 2, cj * 2 - (2 * half - 1))
                idxT_v[j, sl] = jnp.where(j < lenv, adj, SENTINEL)
            return c
        lax.fori_loop(0, L, mask_body, 0)

        def zero_body(e, c):
            for k in range(D // LANES):
                accB[e, pl.ds(k * LANES, LANES)] = jnp.zeros((LANES,),
                                                             jnp.float32)
            return c
        lax.fori_loop(0, chunk, zero_body, 0)

        pltpu.async_copy(emb.at[idxT_v.at[0]], accA, semA)
        pltpu.async_copy(emb.at[_idx(1)], accB, semB, add=True)

        def pair_body(p, c):
            j0 = 2 * p
            pltpu.make_async_copy(emb.at[_idx(j0)], accA, semA).wait()
            pltpu.async_copy(emb.at[_idx(j0)], accA, semA, add=True)
            pltpu.make_async_copy(emb.at[_idx(j0 + 1)], accB, semB).wait()
            pltpu.async_copy(emb.at[_idx(j0 + 1)], accB, semB, add=True)
            return c
        lax.fori_loop(1, L // 2, pair_body, 0)

        pltpu.make_async_copy(emb.at[_idx(0)], accA, semA).wait()
        pltpu.make_async_copy(emb.at[_idx(1)], accB, semB).wait()

        def sum_body(e, c):
            for k in range(D // LANES):
                sl = pl.ds(k * LANES, LANES)
                accA[e, sl] = accA[e, sl] + accB[e, sl]
            return c
        lax.fori_loop(0, chunk, sum_body, 0)

        pltpu.sync_copy(accA, out.at[pl.ds(base, chunk), :])

    do_side(idxT1, len1, out1)
    do_side(idxT2, len2, out2)


def _tr_body(a_ref, b_ref, o_ref):
    o_ref[:, 0:D] = a_ref[...].T
    o_ref[:, D:2 * D] = b_ref[...].T


def _cos_body(s1_ref, s2_ref, l1_ref, l2_ref, o_ref):
    a = s1_ref[...]
    b = s2_ref[...]
    d1 = jnp.maximum(l1_ref[...].astype(jnp.float32), 1.0)
    d2 = jnp.maximum(l2_ref[...].astype(jnp.float32), 1.0)
    dot = jnp.sum(a * b, axis=1, keepdims=True)
    n1 = jnp.sum(a * a, axis=1, keepdims=True)
    n2 = jnp.sum(b * b, axis=1, keepdims=True)
    g = dot / (d1 * d2)
    an = jnp.sqrt(n1) / d1
    bn = jnp.sqrt(n2) / d2
    o_ref[...] = g / jnp.maximum(an * bn, 1e-8)


def kernel(g_idxs1, g_lengths1, g_idxs2, g_lengths2, emb):
    info = plsc.get_sparse_core_info()
    nw = info.num_cores * info.num_subcores
    chunk = B // nw

    idxT1 = g_idxs1.T.astype(jnp.int32)
    idxT2 = g_idxs2.T.astype(jnp.int32)
    len1 = g_lengths1.astype(jnp.int32)
    len2 = g_lengths2.astype(jnp.int32)

    mesh = plsc.VectorSubcoreMesh(core_axis_name="c", subcore_axis_name="s")
    f32 = jnp.float32
    vocab = emb.shape[0]
    tblk = 4096
    half = ((vocab // 2 + tblk - 1) // tblk) * tblk

    sc_fn = pl.kernel(
        functools.partial(_sc_body, info.num_cores, chunk, half),
        compiler_params=pltpu.CompilerParams(use_tc_tiling_on_sc=False),
        out_type=(
            jax.ShapeDtypeStruct((B, D), f32),
            jax.ShapeDtypeStruct((B, D), f32),
        ),
        mesh=mesh,
        scratch_types=[
            pltpu.VMEM((L, chunk), jnp.int32),
            pltpu.VMEM((chunk,), jnp.int32),
            pltpu.VMEM((chunk, D), f32),
            pltpu.VMEM((chunk, D), f32),
            pltpu.SemaphoreType.DMA,
            pltpu.SemaphoreType.DMA,
        ],
    )
    nb = half // tblk
    nb_in = pl.cdiv(vocab, tblk) - 1
    emb128 = pl.pallas_call(
        _tr_body,
        grid=(nb,),
        in_specs=[
            pl.BlockSpec((D, tblk), lambda i: (0, i)),
            pl.BlockSpec((D, tblk), lambda i: (0, jnp.minimum(i + nb,
                                                              nb_in))),
        ],
        out_specs=pl.BlockSpec((tblk, 2 * D), lambda i: (i, 0)),
        out_shape=jax.ShapeDtypeStruct((half, 2 * D), f32),
    )(emb.T, emb.T)
    emb2 = emb128.reshape(2 * half, D)

    s1, s2 = sc_fn(idxT1, len1, idxT2, len2, emb2)

    blk = 2048
    cos = pl.pallas_call(
        _cos_body,
        grid=(B // blk,),
        in_specs=[
            pl.BlockSpec((blk, D), lambda i: (i, 0)),
            pl.BlockSpec((blk, D), lambda i: (i, 0)),
            pl.BlockSpec((blk, 1), lambda i: (i, 0)),
            pl.BlockSpec((blk, 1), lambda i: (i, 0)),
        ],
        out_specs=pl.BlockSpec((blk, 1), lambda i: (i, 0)),
        out_shape=jax.ShapeDtypeStruct((B, 1), f32),
    )(s1, s2, len1.reshape(B, 1), len2.reshape(B, 1))
    return cos.reshape(B)
